# Initial kernel scaffold; baseline (speedup 1.0000x reference)
import jax
import jax.numpy as jnp
from jax.experimental import pallas as pl


def kernel(x, edge_index, edge_weight, W1, b1, eps1, W2, b2, eps2):
    raise NotImplementedError("write your pallas kernel here")


# speedup vs baseline: 12.7588x; 12.7588x over previous
import functools

import jax
import jax.numpy as jnp
from jax import lax
from jax.experimental import pallas as pl
from jax.experimental.pallas import tpu as pltpu
from jax.experimental.pallas import tpu_sc as plsc

N_CORES = 2
N_SUB = 16
N_WORKERS = N_CORES * N_SUB
CHUNK = 128


def _mm1_body(x_ref, w_ref, o_ref):
    o_ref[...] = jnp.dot(x_ref[...], w_ref[...],
                         preferred_element_type=jnp.float32)


def _act_body(p_ref, y_ref, b_ref, s_ref, o_ref):
    z = p_ref[0] + p_ref[1] + s_ref[0, 0] * y_ref[...] + b_ref[...]
    o_ref[...] = jax.nn.sigmoid(z)


def _mm2_body(q_ref, h_ref, w_ref, b_ref, s_ref, o_ref):
    z = q_ref[0] + q_ref[1] + s_ref[0, 0] * h_ref[...]
    o_ref[...] = jnp.dot(z, w_ref[...],
                         preferred_element_type=jnp.float32) + b_ref[...]


@functools.cache
def _make_sc_agg(n_pad, d, n_chunks):
    e_tile = n_chunks * CHUNK
    rows_per_tile = n_pad // N_SUB
    mesh = plsc.VectorSubcoreMesh(core_axis_name="c", subcore_axis_name="s")

    @functools.partial(
        pl.kernel,
        mesh=mesh,
        out_type=jax.ShapeDtypeStruct((N_CORES, n_pad, d), jnp.float32),
        scratch_types=[
            pltpu.VMEM((n_chunks, CHUNK), jnp.int32),
            pltpu.VMEM((n_chunks, CHUNK), jnp.int32),
            pltpu.VMEM((e_tile, d), jnp.float32),
            pltpu.VMEM_SHARED((n_pad, d), jnp.float32),
            pltpu.SemaphoreType.DMA,
            pltpu.SemaphoreType.DMA,
        ],
        compiler_params=pltpu.CompilerParams(use_tc_tiling_on_sc=False),
    )
    def sc_agg(y_hbm, src_hbm, dst_hbm, out_hbm, src_v, dst_v, msgs, acc,
               sem_g, sem_s):
        c = lax.axis_index("c")
        s = lax.axis_index("s")
        wid = s * N_CORES + c

        def zero_body(i, carry):
            msgs[i, :] = jnp.zeros((d,), jnp.float32)
            return carry
        lax.fori_loop(0, rows_per_tile, zero_body, 0)
        pltpu.sync_copy(msgs.at[pl.ds(0, rows_per_tile)],
                        acc.at[pl.ds(s * rows_per_tile, rows_per_tile)])

        pltpu.sync_copy(src_hbm.at[wid], src_v)
        pltpu.sync_copy(dst_hbm.at[wid], dst_v)

        plsc.subcore_barrier()

        def g_fire(j, carry):
            pltpu.async_copy(y_hbm.at[src_v.at[j]],
                             msgs.at[pl.ds(j * CHUNK, CHUNK)], sem_g)
            return carry
        lax.fori_loop(0, n_chunks, g_fire, 0)

        def g_drain_s_fire(j, carry):
            pltpu.make_async_copy(y_hbm.at[src_v.at[j]],
                                  msgs.at[pl.ds(j * CHUNK, CHUNK)],
                                  sem_g).wait()
            pltpu.async_copy(msgs.at[pl.ds(j * CHUNK, CHUNK)],
                             acc.at[dst_v.at[j]], sem_s, add=True)
            return carry
        lax.fori_loop(0, n_chunks, g_drain_s_fire, 0)

        def s_drain(j, carry):
            pltpu.make_async_copy(msgs.at[pl.ds(j * CHUNK, CHUNK)],
                                  acc.at[dst_v.at[j]], sem_s).wait()
            return carry
        lax.fori_loop(0, n_chunks, s_drain, 0)

        plsc.subcore_barrier()

        pltpu.sync_copy(acc.at[pl.ds(s * rows_per_tile, rows_per_tile)],
                        out_hbm.at[c, pl.ds(s * rows_per_tile, rows_per_tile)])

    return sc_agg


def kernel(x, edge_index, edge_weight, W1, b1, eps1, W2, b2, eps2):
    n, d_in = x.shape
    d_hid = W1.shape[1]
    d_out = W2.shape[1]
    n_edges = edge_index.shape[1]

    e_tile = -(-n_edges // (N_WORKERS * CHUNK)) * CHUNK
    n_chunks = e_tile // CHUNK
    e_pad = N_WORKERS * e_tile - n_edges
    n_pad = -(-(n + 1) // (N_SUB * 8)) * (N_SUB * 8)

    src = edge_index[0].astype(jnp.int32)
    dst = edge_index[1].astype(jnp.int32)
    src_p = jnp.concatenate([src, jnp.zeros((e_pad,), jnp.int32)])
    dst_p = jnp.concatenate([dst, jnp.full((e_pad,), n, jnp.int32)])
    src_p = src_p.reshape(N_WORKERS, n_chunks, CHUNK)
    dst_p = dst_p.reshape(N_WORKERS, n_chunks, CHUNK)

    sc_agg = _make_sc_agg(n_pad, d_hid, n_chunks)

    mb = 1000
    grid = (n // mb,)

    y = pl.pallas_call(
        _mm1_body,
        grid=grid,
        in_specs=[pl.BlockSpec((mb, d_in), lambda i: (i, 0)),
                  pl.BlockSpec((d_in, d_hid), lambda i: (0, 0))],
        out_specs=pl.BlockSpec((mb, d_hid), lambda i: (i, 0)),
        out_shape=jax.ShapeDtypeStruct((n, d_hid), jnp.float32),
    )(x, W1)

    p = sc_agg(y, src_p, dst_p)

    scale1 = (1.0 + eps1).astype(jnp.float32).reshape(1, 1)
    scale2 = (1.0 + eps2).astype(jnp.float32).reshape(1, 1)

    h = pl.pallas_call(
        _act_body,
        grid=grid,
        in_specs=[pl.BlockSpec((N_CORES, mb, d_hid), lambda i: (0, i, 0)),
                  pl.BlockSpec((mb, d_hid), lambda i: (i, 0)),
                  pl.BlockSpec((1, d_hid), lambda i: (0, 0)),
                  pl.BlockSpec((1, 1), lambda i: (0, 0))],
        out_specs=pl.BlockSpec((mb, d_hid), lambda i: (i, 0)),
        out_shape=jax.ShapeDtypeStruct((n, d_hid), jnp.float32),
    )(p, y, b1.reshape(1, d_hid), scale1)

    q = sc_agg(h, src_p, dst_p)

    out = pl.pallas_call(
        _mm2_body,
        grid=grid,
        in_specs=[pl.BlockSpec((N_CORES, mb, d_hid), lambda i: (0, i, 0)),
                  pl.BlockSpec((mb, d_hid), lambda i: (i, 0)),
                  pl.BlockSpec((d_hid, d_out), lambda i: (0, 0)),
                  pl.BlockSpec((1, d_out), lambda i: (0, 0)),
                  pl.BlockSpec((1, 1), lambda i: (0, 0))],
        out_specs=pl.BlockSpec((mb, d_out), lambda i: (i, 0)),
        out_shape=jax.ShapeDtypeStruct((n, d_out), jnp.float32),
    )(q, h, W2, b2.reshape(1, d_out), scale2)

    return out


# speedup vs baseline: 17.8530x; 1.3993x over previous
import functools

import jax
import jax.numpy as jnp
from jax import lax
from jax.experimental import pallas as pl
from jax.experimental.pallas import tpu as pltpu
from jax.experimental.pallas import tpu_sc as plsc

N_CORES = 2
N_SUB = 16
N_WORKERS = N_CORES * N_SUB
CHUNK = 128


def _mm1_body(x_ref, w_ref, o_ref):
    o_ref[...] = jnp.dot(x_ref[...], w_ref[...],
                         preferred_element_type=jnp.float32)


def _act_body(p_ref, y_ref, b_ref, s_ref, o_ref):
    z = p_ref[0] + p_ref[1] + s_ref[0, 0] * y_ref[...] + b_ref[...]
    o_ref[...] = jax.nn.sigmoid(z)


def _mm2_body(q_ref, h_ref, w_ref, b_ref, s_ref, o_ref):
    z = q_ref[0] + q_ref[1] + s_ref[0, 0] * h_ref[...]
    o_ref[...] = jnp.dot(z, w_ref[...],
                         preferred_element_type=jnp.float32) + b_ref[...]


@functools.cache
def _make_sc_agg(n_pad, d, n_chunks):
    e_tile = n_chunks * CHUNK
    rows_per_tile = n_pad // N_SUB
    mesh = plsc.VectorSubcoreMesh(core_axis_name="c", subcore_axis_name="s")

    @functools.partial(
        pl.kernel,
        mesh=mesh,
        out_type=jax.ShapeDtypeStruct((N_CORES, n_pad, d), jnp.float32),
        scratch_types=[
            pltpu.VMEM((n_chunks, CHUNK), jnp.int32),
            pltpu.VMEM((n_chunks, CHUNK), jnp.int32),
            pltpu.VMEM((e_tile, d), jnp.float32),
            pltpu.VMEM_SHARED((n_pad, d), jnp.float32),
            pltpu.SemaphoreType.DMA,
            pltpu.SemaphoreType.DMA,
        ],
        compiler_params=pltpu.CompilerParams(use_tc_tiling_on_sc=False),
    )
    def sc_agg(y_hbm, src_hbm, dst_hbm, out_hbm, src_v, dst_v, msgs, acc,
               sem_g, sem_s):
        c = lax.axis_index("c")
        s = lax.axis_index("s")
        wid = s * N_CORES + c

        def zero_body(i, carry):
            msgs[i, :] = jnp.zeros((d,), jnp.float32)
            return carry
        lax.fori_loop(0, rows_per_tile, zero_body, 0)
        pltpu.sync_copy(msgs.at[pl.ds(0, rows_per_tile)],
                        acc.at[pl.ds(s * rows_per_tile, rows_per_tile)])

        pltpu.sync_copy(src_hbm.at[wid], src_v)
        pltpu.sync_copy(dst_hbm.at[wid], dst_v)

        plsc.subcore_barrier()

        def g_fire(j, carry):
            pltpu.async_copy(y_hbm.at[src_v.at[j]],
                             msgs.at[pl.ds(j * CHUNK, CHUNK)], sem_g)
            return carry
        lax.fori_loop(0, n_chunks, g_fire, 0)

        def g_drain_s_fire(j, carry):
            pltpu.make_async_copy(y_hbm.at[src_v.at[j]],
                                  msgs.at[pl.ds(j * CHUNK, CHUNK)],
                                  sem_g).wait()
            pltpu.async_copy(msgs.at[pl.ds(j * CHUNK, CHUNK)],
                             acc.at[dst_v.at[j]], sem_s, add=True)
            return carry
        lax.fori_loop(0, n_chunks, g_drain_s_fire, 0)

        def s_drain(j, carry):
            pltpu.make_async_copy(msgs.at[pl.ds(j * CHUNK, CHUNK)],
                                  acc.at[dst_v.at[j]], sem_s).wait()
            return carry
        lax.fori_loop(0, n_chunks, s_drain, 0)

        plsc.subcore_barrier()

        pltpu.sync_copy(acc.at[pl.ds(s * rows_per_tile, rows_per_tile)],
                        out_hbm.at[c, pl.ds(s * rows_per_tile, rows_per_tile)])

    return sc_agg


def kernel(x, edge_index, edge_weight, W1, b1, eps1, W2, b2, eps2):
    n, d_in = x.shape
    d_hid = W1.shape[1]
    d_out = W2.shape[1]
    n_edges = edge_index.shape[1]

    e_tile = -(-n_edges // (N_WORKERS * CHUNK)) * CHUNK
    n_chunks = e_tile // CHUNK
    e_pad = N_WORKERS * e_tile - n_edges
    n_pad = -(-(n + 1) // (N_SUB * 8)) * (N_SUB * 8)

    src = edge_index[0].astype(jnp.int32)
    dst = edge_index[1].astype(jnp.int32)
    pad_iota = jnp.arange(e_pad, dtype=jnp.int32)
    src_p = jnp.concatenate([src, pad_iota % n])
    dst_p = jnp.concatenate([dst, n + pad_iota % (n_pad - n)])
    src_p = src_p.reshape(N_WORKERS, n_chunks, CHUNK)
    dst_p = dst_p.reshape(N_WORKERS, n_chunks, CHUNK)

    sc_agg = _make_sc_agg(n_pad, d_hid, n_chunks)

    mb = 1000
    grid = (n // mb,)

    y = pl.pallas_call(
        _mm1_body,
        grid=grid,
        in_specs=[pl.BlockSpec((mb, d_in), lambda i: (i, 0)),
                  pl.BlockSpec((d_in, d_hid), lambda i: (0, 0))],
        out_specs=pl.BlockSpec((mb, d_hid), lambda i: (i, 0)),
        out_shape=jax.ShapeDtypeStruct((n, d_hid), jnp.float32),
    )(x, W1)

    p = sc_agg(y, src_p, dst_p)

    scale1 = (1.0 + eps1).astype(jnp.float32).reshape(1, 1)
    scale2 = (1.0 + eps2).astype(jnp.float32).reshape(1, 1)

    h = pl.pallas_call(
        _act_body,
        grid=grid,
        in_specs=[pl.BlockSpec((N_CORES, mb, d_hid), lambda i: (0, i, 0)),
                  pl.BlockSpec((mb, d_hid), lambda i: (i, 0)),
                  pl.BlockSpec((1, d_hid), lambda i: (0, 0)),
                  pl.BlockSpec((1, 1), lambda i: (0, 0))],
        out_specs=pl.BlockSpec((mb, d_hid), lambda i: (i, 0)),
        out_shape=jax.ShapeDtypeStruct((n, d_hid), jnp.float32),
    )(p, y, b1.reshape(1, d_hid), scale1)

    q = sc_agg(h, src_p, dst_p)

    out = pl.pallas_call(
        _mm2_body,
        grid=grid,
        in_specs=[pl.BlockSpec((N_CORES, mb, d_hid), lambda i: (0, i, 0)),
                  pl.BlockSpec((mb, d_hid), lambda i: (i, 0)),
                  pl.BlockSpec((d_hid, d_out), lambda i: (0, 0)),
                  pl.BlockSpec((1, d_out), lambda i: (0, 0)),
                  pl.BlockSpec((1, 1), lambda i: (0, 0))],
        out_specs=pl.BlockSpec((mb, d_out), lambda i: (i, 0)),
        out_shape=jax.ShapeDtypeStruct((n, d_out), jnp.float32),
    )(q, h, W2, b2.reshape(1, d_out), scale2)

    return out
